# Initial kernel scaffold; baseline (speedup 1.0000x reference)
import jax
import jax.numpy as jnp
from jax.experimental import pallas as pl


def kernel(x, edge_index, W1, b1, W2, b2):
    raise NotImplementedError("write your pallas kernel here")


# speedup vs baseline: 12.9822x; 12.9822x over previous
import functools

import jax
import jax.numpy as jnp
from jax import lax
from jax.experimental import pallas as pl
from jax.experimental.pallas import tpu as pltpu
from jax.experimental.pallas import tpu_sc as plsc

N = 10000
D = 128
E = 320000
NC = 2
NS = 16
NW = NC * NS
EPW = E // NW
C = 80
NCHUNK = EPW // C
NP = NS * 640
RPS = NP // NS

_mesh = plsc.VectorSubcoreMesh(core_axis_name="c", subcore_axis_name="s")


@functools.partial(
    pl.kernel,
    out_type=(jax.ShapeDtypeStruct((NP,), jnp.float32),
              jax.ShapeDtypeStruct((NP,), jnp.float32)),
    mesh=_mesh,
    scratch_types=[
        pltpu.VMEM_SHARED((NP,), jnp.float32),
        pltpu.VMEM((C,), jnp.int32),
        pltpu.VMEM((C,), jnp.float32),
        pltpu.VMEM((RPS,), jnp.float32),
    ],
)
def _deg_kernel(dst_hbm, ones_hbm, out0_hbm, out1_hbm,
                deg_sp, idx_v, ones_v, buf_v):
    c = lax.axis_index("c")
    s = lax.axis_index("s")
    w = c * NS + s
    pltpu.sync_copy(ones_hbm, buf_v)
    pltpu.sync_copy(ones_hbm.at[pl.ds(0, C)], ones_v)
    pltpu.sync_copy(buf_v, deg_sp.at[pl.ds(s * RPS, RPS)])
    plsc.subcore_barrier()

    def body(i, carry):
        base = pl.multiple_of(w * EPW + i * C, C)
        pltpu.sync_copy(dst_hbm.at[pl.ds(base, C)], idx_v)
        pltpu.sync_copy(ones_v, deg_sp.at[idx_v], add=True)
        return carry

    lax.fori_loop(0, NCHUNK, body, 0)
    plsc.subcore_barrier()
    pltpu.sync_copy(deg_sp.at[pl.ds(s * RPS, RPS)], buf_v)
    @pl.when(c == 0)
    def _():
        pltpu.sync_copy(buf_v, out0_hbm.at[pl.ds(s * RPS, RPS)])
    @pl.when(c == 1)
    def _():
        pltpu.sync_copy(buf_v, out1_hbm.at[pl.ds(s * RPS, RPS)])


@functools.partial(
    pl.kernel,
    out_type=(jax.ShapeDtypeStruct((NP, D), jnp.float32),
              jax.ShapeDtypeStruct((NP, D), jnp.float32)),
    mesh=_mesh,
    scratch_types=[
        pltpu.VMEM_SHARED((NP, D), jnp.float32),
        pltpu.VMEM((C,), jnp.int32),
        pltpu.VMEM((C,), jnp.int32),
        pltpu.VMEM((C, D), jnp.float32),
        pltpu.VMEM((128, D), jnp.float32),
        pltpu.SemaphoreType.DMA,
    ],
)
def _agg_kernel(g_hbm, src_hbm, dst_hbm, zeros_hbm, out0_hbm, out1_hbm,
                acc_sp, src_v, dst_v, rows_v, zbuf_v, sem):
    c = lax.axis_index("c")
    s = lax.axis_index("s")
    w = c * NS + s
    pltpu.sync_copy(zeros_hbm, zbuf_v)
    for j in range(5):
        pltpu.sync_copy(zbuf_v, acc_sp.at[pl.ds(s * RPS + j * 128, 128)])
    plsc.subcore_barrier()

    def body(i, carry):
        base = pl.multiple_of(w * EPW + i * C, C)
        pltpu.sync_copy(src_hbm.at[pl.ds(base, C)], src_v)
        pltpu.sync_copy(dst_hbm.at[pl.ds(base, C)], dst_v)
        pltpu.async_copy(g_hbm.at[src_v], rows_v, sem).wait()
        pltpu.sync_copy(rows_v, acc_sp.at[dst_v], add=True)
        return carry

    lax.fori_loop(0, NCHUNK, body, 0)
    plsc.subcore_barrier()
    for j in range(5):
        pltpu.sync_copy(acc_sp.at[pl.ds(s * RPS + j * 128, 128)], zbuf_v)
        @pl.when(c == 0)
        def _():
            pltpu.sync_copy(zbuf_v, out0_hbm.at[pl.ds(s * RPS + j * 128, 128)])
        @pl.when(c == 1)
        def _():
            pltpu.sync_copy(zbuf_v, out1_hbm.at[pl.ds(s * RPS + j * 128, 128)])


_BLK = 1024
_GRID = NP // _BLK


def _row_spec():
    return pl.BlockSpec((_BLK, D), lambda i: (i, 0))


def _col_spec():
    return pl.BlockSpec((_BLK, 1), lambda i: (i, 0))


def _full_spec():
    return pl.BlockSpec((D, D), lambda i: (0, 0))


def _bias_spec():
    return pl.BlockSpec((1, D), lambda i: (0, 0))


def _tc1_body(d0_ref, d1_ref, x_ref, w1_ref, g_ref, dinv_ref):
    deg = d0_ref[...] + d1_ref[...] - 1.0
    dinv = lax.rsqrt(deg)
    h = jnp.dot(x_ref[...], w1_ref[...], preferred_element_type=jnp.float32)
    g_ref[...] = h * dinv
    dinv_ref[...] = dinv


_tc1 = pl.pallas_call(
    _tc1_body,
    grid=(_GRID,),
    in_specs=[_col_spec(), _col_spec(), _row_spec(), _full_spec()],
    out_specs=[_row_spec(), _col_spec()],
    out_shape=(jax.ShapeDtypeStruct((NP, D), jnp.float32),
               jax.ShapeDtypeStruct((NP, 1), jnp.float32)),
)


def _tc2_body(p0_ref, p1_ref, g1_ref, dinv_ref, b1_ref, w2_ref, g2_ref):
    dinv = dinv_ref[...]
    a = dinv * (p0_ref[...] + p1_ref[...] + g1_ref[...]) + b1_ref[...]
    t = 0.5 * a * (1.0 + lax.erf(a * 0.7071067811865476))
    g2_ref[...] = jnp.dot(t, w2_ref[...],
                          preferred_element_type=jnp.float32) * dinv


_tc2 = pl.pallas_call(
    _tc2_body,
    grid=(_GRID,),
    in_specs=[_row_spec(), _row_spec(), _row_spec(), _col_spec(),
              _bias_spec(), _full_spec()],
    out_specs=_row_spec(),
    out_shape=jax.ShapeDtypeStruct((NP, D), jnp.float32),
)


def _tc3_body(q0_ref, q1_ref, g2_ref, dinv_ref, b2_ref, out_ref):
    out_ref[...] = (dinv_ref[...] * (q0_ref[...] + q1_ref[...] + g2_ref[...])
                    + b2_ref[...])


_tc3 = pl.pallas_call(
    _tc3_body,
    grid=(_GRID,),
    in_specs=[_row_spec(), _row_spec(), _row_spec(), _col_spec(),
              _bias_spec()],
    out_specs=_row_spec(),
    out_shape=jax.ShapeDtypeStruct((NP, D), jnp.float32),
)


def kernel(x, edge_index, W1, b1, W2, b2):
    src = edge_index[0]
    dst = edge_index[1]
    x_p = jnp.pad(x, ((0, NP - N), (0, 0)))
    ones_c = jnp.ones((RPS,), jnp.float32)
    zeros_c = jnp.zeros((128, D), jnp.float32)

    d0, d1 = _deg_kernel(dst, ones_c)
    g1, dinv = _tc1(d0.reshape(NP, 1), d1.reshape(NP, 1), x_p, W1)
    p0, p1 = _agg_kernel(g1, src, dst, zeros_c)
    g2 = _tc2(p0, p1, g1, dinv, b1.reshape(1, D), W2)
    q0, q1 = _agg_kernel(g2, src, dst, zeros_c)
    out = _tc3(q0, q1, g2, dinv, b2.reshape(1, D))
    return out[:N]


# speedup vs baseline: 24.6265x; 1.8969x over previous
import functools

import jax
import jax.numpy as jnp
from jax import lax
from jax.experimental import pallas as pl
from jax.experimental.pallas import tpu as pltpu
from jax.experimental.pallas import tpu_sc as plsc

N = 10000
D = 128
E = 320000
NC = 2
NS = 16
NW = NC * NS
EPW = E // NW
C = 80
NCHUNK = EPW // C
NP = NS * 640
RPS = NP // NS
NBUF = 2

_mesh = plsc.VectorSubcoreMesh(core_axis_name="c", subcore_axis_name="s")


@functools.partial(
    pl.kernel,
    out_type=(jax.ShapeDtypeStruct((NP,), jnp.float32),
              jax.ShapeDtypeStruct((NP,), jnp.float32)),
    mesh=_mesh,
    scratch_types=[
        pltpu.VMEM_SHARED((NP,), jnp.float32),
        pltpu.VMEM((NCHUNK, C), jnp.int32),
        pltpu.VMEM((C,), jnp.float32),
        pltpu.VMEM((RPS,), jnp.float32),
        pltpu.SemaphoreType.DMA,
        pltpu.SemaphoreType.DMA,
        pltpu.SemaphoreType.DMA,
        pltpu.SemaphoreType.DMA,
    ],
)
def _deg_kernel(dstr_hbm, ones_hbm, out0_hbm, out1_hbm,
                deg_sp, dsts_v, ones_v, buf_v, s0, s1, s2, s3):
    c = lax.axis_index("c")
    s = lax.axis_index("s")
    w = c * NS + s
    sems = [s0, s1, s2, s3]
    pltpu.sync_copy(ones_hbm, buf_v)
    pltpu.sync_copy(ones_hbm.at[pl.ds(0, C)], ones_v)
    pltpu.sync_copy(dstr_hbm.at[w], dsts_v)
    pltpu.sync_copy(buf_v, deg_sp.at[pl.ds(s * RPS, RPS)])
    plsc.subcore_barrier()

    def swait(sem):
        pltpu.make_async_copy(ones_v, deg_sp.at[dsts_v.at[0]], sem).wait()

    def body(j, carry):
        for b in range(NBUF):
            @pl.when(j > 0)
            def _():
                swait(sems[b])
            pltpu.async_copy(ones_v, deg_sp.at[dsts_v.at[j * NBUF + b]],
                             sems[b], add=True)
        return carry

    lax.fori_loop(0, NCHUNK // NBUF, body, 0)
    swait(sems[0])
    pltpu.async_copy(ones_v, deg_sp.at[dsts_v.at[NCHUNK - 1]], s0, add=True)
    for b in range(NBUF):
        swait(sems[b])
    plsc.subcore_barrier()
    pltpu.sync_copy(deg_sp.at[pl.ds(s * RPS, RPS)], buf_v)
    @pl.when(c == 0)
    def _():
        pltpu.sync_copy(buf_v, out0_hbm.at[pl.ds(s * RPS, RPS)])
    @pl.when(c == 1)
    def _():
        pltpu.sync_copy(buf_v, out1_hbm.at[pl.ds(s * RPS, RPS)])


@functools.partial(
    pl.kernel,
    out_type=(jax.ShapeDtypeStruct((NP, D), jnp.float32),
              jax.ShapeDtypeStruct((NP, D), jnp.float32)),
    mesh=_mesh,
    scratch_types=[
        pltpu.VMEM_SHARED((NP, D), jnp.float32),
        pltpu.VMEM((EPW,), jnp.int32),
        pltpu.VMEM((NCHUNK, C), jnp.int32),
        [pltpu.VMEM((C, D), jnp.float32)] * NBUF,
        [pltpu.SemaphoreType.DMA] * NBUF,
        [pltpu.SemaphoreType.DMA] * NBUF,
    ],
)
def _agg_kernel(g_hbm, srcf_hbm, dstr_hbm, zeros_hbm, out0_hbm, out1_hbm,
                acc_sp, srcs_v, dsts_v, rows, gsems, ssems):
    c = lax.axis_index("c")
    s = lax.axis_index("s")
    w = c * NS + s
    pltpu.sync_copy(zeros_hbm, rows[0])
    for j in range(8):
        pltpu.sync_copy(rows[0], acc_sp.at[pl.ds(s * RPS + j * C, C)])
    pltpu.sync_copy(srcf_hbm.at[w], srcs_v)
    pltpu.sync_copy(dstr_hbm.at[w], dsts_v)
    plsc.subcore_barrier()

    def src_at(i):
        return srcs_v.at[pl.ds(pl.multiple_of(i * C, C), C)]

    def gwait(b):
        pltpu.make_async_copy(g_hbm.at[src_at(0)], rows[b], gsems[b]).wait()

    def swait(b):
        pltpu.make_async_copy(rows[b], acc_sp.at[dsts_v.at[0]],
                              ssems[b]).wait()

    for b in range(NBUF):
        pltpu.async_copy(g_hbm.at[src_at(b)], rows[b], gsems[b])

    def body(j, carry):
        i0 = j * NBUF
        for b in range(NBUF):
            gwait(b)
            pltpu.async_copy(rows[b], acc_sp.at[dsts_v.at[i0 + b]],
                             ssems[b], add=True)
        for b in range(NBUF):
            nxt = i0 + NBUF + b
            @pl.when(nxt < NCHUNK)
            def _():
                swait(b)
                pltpu.async_copy(g_hbm.at[src_at(nxt)], rows[b], gsems[b])
        return carry

    lax.fori_loop(0, NCHUNK // NBUF, body, 0)
    gwait(0)
    pltpu.async_copy(rows[0], acc_sp.at[dsts_v.at[NCHUNK - 1]],
                     ssems[0], add=True)
    for b in range(NBUF):
        swait(b)
    plsc.subcore_barrier()
    for j in range(8):
        pltpu.sync_copy(acc_sp.at[pl.ds(s * RPS + j * C, C)], rows[0])
        @pl.when(c == 0)
        def _():
            pltpu.sync_copy(rows[0], out0_hbm.at[pl.ds(s * RPS + j * C, C)])
        @pl.when(c == 1)
        def _():
            pltpu.sync_copy(rows[0], out1_hbm.at[pl.ds(s * RPS + j * C, C)])


_BLK = 1024
_GRID = NP // _BLK


def _row_spec():
    return pl.BlockSpec((_BLK, D), lambda i: (i, 0))


def _col_spec():
    return pl.BlockSpec((_BLK, 1), lambda i: (i, 0))


def _full_spec():
    return pl.BlockSpec((D, D), lambda i: (0, 0))


def _bias_spec():
    return pl.BlockSpec((1, D), lambda i: (0, 0))


def _tc1_body(d0_ref, d1_ref, x_ref, w1_ref, g_ref, dinv_ref):
    deg = d0_ref[...] + d1_ref[...] - 1.0
    dinv = lax.rsqrt(deg)
    h = jnp.dot(x_ref[...], w1_ref[...], preferred_element_type=jnp.float32)
    g_ref[...] = h * dinv
    dinv_ref[...] = dinv


_tc1 = pl.pallas_call(
    _tc1_body,
    grid=(_GRID,),
    in_specs=[_col_spec(), _col_spec(), _row_spec(), _full_spec()],
    out_specs=[_row_spec(), _col_spec()],
    out_shape=(jax.ShapeDtypeStruct((NP, D), jnp.float32),
               jax.ShapeDtypeStruct((NP, 1), jnp.float32)),
)


def _tc2_body(p0_ref, p1_ref, g1_ref, dinv_ref, b1_ref, w2_ref, g2_ref):
    dinv = dinv_ref[...]
    a = dinv * (p0_ref[...] + p1_ref[...] + g1_ref[...]) + b1_ref[...]
    t = 0.5 * a * (1.0 + lax.erf(a * 0.7071067811865476))
    g2_ref[...] = jnp.dot(t, w2_ref[...],
                          preferred_element_type=jnp.float32) * dinv


_tc2 = pl.pallas_call(
    _tc2_body,
    grid=(_GRID,),
    in_specs=[_row_spec(), _row_spec(), _row_spec(), _col_spec(),
              _bias_spec(), _full_spec()],
    out_specs=_row_spec(),
    out_shape=jax.ShapeDtypeStruct((NP, D), jnp.float32),
)


def _tc3_body(q0_ref, q1_ref, g2_ref, dinv_ref, b2_ref, out_ref):
    out_ref[...] = (dinv_ref[...] * (q0_ref[...] + q1_ref[...] + g2_ref[...])
                    + b2_ref[...])


_tc3 = pl.pallas_call(
    _tc3_body,
    grid=(_GRID,),
    in_specs=[_row_spec(), _row_spec(), _row_spec(), _col_spec(),
              _bias_spec()],
    out_specs=_row_spec(),
    out_shape=jax.ShapeDtypeStruct((NP, D), jnp.float32),
)


def kernel(x, edge_index, W1, b1, W2, b2):
    src_f = edge_index[0].reshape(NW, EPW)
    dst_r = edge_index[1].reshape(NW, NCHUNK, C)
    x_p = jnp.pad(x, ((0, NP - N), (0, 0)))
    ones_c = jnp.ones((RPS,), jnp.float32)
    zeros_c = jnp.zeros((C, D), jnp.float32)

    d0, d1 = _deg_kernel(dst_r, ones_c)
    g1, dinv = _tc1(d0.reshape(NP, 1), d1.reshape(NP, 1), x_p, W1)
    p0, p1 = _agg_kernel(g1, src_f, dst_r, zeros_c)
    g2 = _tc2(p0, p1, g1, dinv, b1.reshape(1, D), W2)
    q0, q1 = _agg_kernel(g2, src_f, dst_r, zeros_c)
    out = _tc3(q0, q1, g2, dinv, b2.reshape(1, D))
    return out[:N]


# speedup vs baseline: 25.2805x; 1.0266x over previous
import functools

import jax
import jax.numpy as jnp
from jax import lax
from jax.experimental import pallas as pl
from jax.experimental.pallas import tpu as pltpu
from jax.experimental.pallas import tpu_sc as plsc

N = 10000
D = 128
E = 320000
NC = 2
NS = 16
NW = NC * NS
EPW = E // NW
C = 80
NCHUNK = EPW // C
NP = NS * 640
RPS = NP // NS
NBUF = 2

_mesh = plsc.VectorSubcoreMesh(core_axis_name="c", subcore_axis_name="s")


@functools.partial(
    pl.kernel,
    out_type=(jax.ShapeDtypeStruct((NP,), jnp.float32),
              jax.ShapeDtypeStruct((NP,), jnp.float32)),
    mesh=_mesh,
    scratch_types=[
        pltpu.VMEM_SHARED((NP,), jnp.float32),
        pltpu.VMEM((NCHUNK, C), jnp.int32),
        pltpu.VMEM((C,), jnp.float32),
        pltpu.VMEM((RPS,), jnp.float32),
        pltpu.SemaphoreType.DMA,
        pltpu.SemaphoreType.DMA,
        pltpu.SemaphoreType.DMA,
        pltpu.SemaphoreType.DMA,
    ],
)
def _deg_kernel(dstr_hbm, ones_hbm, out0_hbm, out1_hbm,
                deg_sp, dsts_v, ones_v, buf_v, s0, s1, s2, s3):
    c = lax.axis_index("c")
    s = lax.axis_index("s")
    w = c * NS + s
    sems = [s0, s1, s2, s3]
    pltpu.sync_copy(ones_hbm, buf_v)
    pltpu.sync_copy(ones_hbm.at[pl.ds(0, C)], ones_v)
    pltpu.sync_copy(dstr_hbm.at[w], dsts_v)
    pltpu.sync_copy(buf_v, deg_sp.at[pl.ds(s * RPS, RPS)])
    plsc.subcore_barrier()

    def swait(sem):
        pltpu.make_async_copy(ones_v, deg_sp.at[dsts_v.at[0]], sem).wait()

    def body(j, carry):
        for b in range(NBUF):
            @pl.when(j > 0)
            def _():
                swait(sems[b])
            pltpu.async_copy(ones_v, deg_sp.at[dsts_v.at[j * NBUF + b]],
                             sems[b], add=True)
        return carry

    lax.fori_loop(0, NCHUNK // NBUF, body, 0)
    swait(sems[0])
    pltpu.async_copy(ones_v, deg_sp.at[dsts_v.at[NCHUNK - 1]], s0, add=True)
    for b in range(NBUF):
        swait(sems[b])
    plsc.subcore_barrier()
    pltpu.sync_copy(deg_sp.at[pl.ds(s * RPS, RPS)], buf_v)
    @pl.when(c == 0)
    def _():
        pltpu.sync_copy(buf_v, out0_hbm.at[pl.ds(s * RPS, RPS)])
    @pl.when(c == 1)
    def _():
        pltpu.sync_copy(buf_v, out1_hbm.at[pl.ds(s * RPS, RPS)])


@functools.partial(
    pl.kernel,
    out_type=(jax.ShapeDtypeStruct((NP, D), jnp.float32),
              jax.ShapeDtypeStruct((NP, D), jnp.float32)),
    mesh=_mesh,
    scratch_types=[
        pltpu.VMEM_SHARED((NP, D), jnp.float32),
        pltpu.VMEM((EPW,), jnp.int32),
        pltpu.VMEM((NCHUNK, C), jnp.int32),
        [pltpu.VMEM((C, D), jnp.float32)] * NBUF,
        [pltpu.SemaphoreType.DMA] * NBUF,
        [pltpu.SemaphoreType.DMA] * NBUF,
        pltpu.SemaphoreType.DMA,
    ],
)
def _agg_kernel(g_hbm, srcf_hbm, dstr_hbm, zeros_hbm, out0_hbm, out1_hbm,
                acc_sp, srcs_v, dsts_v, rows, gsems, ssems, zsem):
    c = lax.axis_index("c")
    s = lax.axis_index("s")
    w = c * NS + s
    pltpu.sync_copy(srcf_hbm.at[w], srcs_v)
    pltpu.sync_copy(dstr_hbm.at[w], dsts_v)
    pltpu.sync_copy(zeros_hbm, rows[0])
    for j in range(8):
        pltpu.async_copy(rows[0], acc_sp.at[pl.ds(s * RPS + j * C, C)], zsem)
    for j in range(8):
        pltpu.make_async_copy(rows[0], acc_sp.at[pl.ds(0, C)], zsem).wait()
    plsc.subcore_barrier()

    def src_at(i):
        return srcs_v.at[pl.ds(pl.multiple_of(i * C, C), C)]

    def gwait(b):
        pltpu.make_async_copy(g_hbm.at[src_at(0)], rows[b], gsems[b]).wait()

    def swait(b):
        pltpu.make_async_copy(rows[b], acc_sp.at[dsts_v.at[0]],
                              ssems[b]).wait()

    for b in range(NBUF):
        pltpu.async_copy(g_hbm.at[src_at(b)], rows[b], gsems[b])

    def body(j, carry):
        i0 = j * NBUF
        for b in range(NBUF):
            gwait(b)
            pltpu.async_copy(rows[b], acc_sp.at[dsts_v.at[i0 + b]],
                             ssems[b], add=True)
        for b in range(NBUF):
            nxt = i0 + NBUF + b
            @pl.when(nxt < NCHUNK)
            def _():
                swait(b)
                pltpu.async_copy(g_hbm.at[src_at(nxt)], rows[b], gsems[b])
        return carry

    lax.fori_loop(0, NCHUNK // NBUF, body, 0)
    gwait(0)
    pltpu.async_copy(rows[0], acc_sp.at[dsts_v.at[NCHUNK - 1]],
                     ssems[0], add=True)
    for b in range(NBUF):
        swait(b)
    plsc.subcore_barrier()
    def dread(j, b):
        pltpu.async_copy(acc_sp.at[pl.ds(s * RPS + j * C, C)], rows[b],
                         gsems[b])

    def dwrite(j, b):
        pltpu.make_async_copy(acc_sp.at[pl.ds(0, C)], rows[b],
                              gsems[b]).wait()
        @pl.when(c == 0)
        def _():
            pltpu.async_copy(rows[b], out0_hbm.at[pl.ds(s * RPS + j * C, C)],
                             ssems[b])
        @pl.when(c == 1)
        def _():
            pltpu.async_copy(rows[b], out1_hbm.at[pl.ds(s * RPS + j * C, C)],
                             ssems[b])

    def dwwait(b):
        pltpu.make_async_copy(rows[b], out0_hbm.at[pl.ds(0, C)],
                              ssems[b]).wait()

    dread(0, 0)
    dread(1, 1)
    for j in range(8):
        b = j % 2
        dwrite(j, b)
        if j + 2 < 8:
            dwwait(b)
            dread(j + 2, b)
    dwwait(0)
    dwwait(1)


_BLK = 2048
_GRID = NP // _BLK


def _row_spec():
    return pl.BlockSpec((_BLK, D), lambda i: (i, 0))


def _col_spec():
    return pl.BlockSpec((_BLK, 1), lambda i: (i, 0))


def _full_spec():
    return pl.BlockSpec((D, D), lambda i: (0, 0))


def _bias_spec():
    return pl.BlockSpec((1, D), lambda i: (0, 0))


def _tc1_body(d0_ref, d1_ref, x_ref, w1_ref, g_ref, dinv_ref):
    deg = d0_ref[...] + d1_ref[...] - 1.0
    dinv = lax.rsqrt(deg)
    h = jnp.dot(x_ref[...], w1_ref[...], preferred_element_type=jnp.float32)
    g_ref[...] = h * dinv
    dinv_ref[...] = dinv


_tc1 = pl.pallas_call(
    _tc1_body,
    grid=(_GRID,),
    in_specs=[_col_spec(), _col_spec(), _row_spec(), _full_spec()],
    out_specs=[_row_spec(), _col_spec()],
    out_shape=(jax.ShapeDtypeStruct((NP, D), jnp.float32),
               jax.ShapeDtypeStruct((NP, 1), jnp.float32)),
)


def _tc2_body(p0_ref, p1_ref, g1_ref, dinv_ref, b1_ref, w2_ref, g2_ref):
    dinv = dinv_ref[...]
    a = dinv * (p0_ref[...] + p1_ref[...] + g1_ref[...]) + b1_ref[...]
    t = 0.5 * a * (1.0 + lax.erf(a * 0.7071067811865476))
    g2_ref[...] = jnp.dot(t, w2_ref[...],
                          preferred_element_type=jnp.float32) * dinv


_tc2 = pl.pallas_call(
    _tc2_body,
    grid=(_GRID,),
    in_specs=[_row_spec(), _row_spec(), _row_spec(), _col_spec(),
              _bias_spec(), _full_spec()],
    out_specs=_row_spec(),
    out_shape=jax.ShapeDtypeStruct((NP, D), jnp.float32),
)


def _tc3_body(q0_ref, q1_ref, g2_ref, dinv_ref, b2_ref, out_ref):
    out_ref[...] = (dinv_ref[...] * (q0_ref[...] + q1_ref[...] + g2_ref[...])
                    + b2_ref[...])


_tc3 = pl.pallas_call(
    _tc3_body,
    grid=(_GRID,),
    in_specs=[_row_spec(), _row_spec(), _row_spec(), _col_spec(),
              _bias_spec()],
    out_specs=_row_spec(),
    out_shape=jax.ShapeDtypeStruct((NP, D), jnp.float32),
)


def kernel(x, edge_index, W1, b1, W2, b2):
    src_f = edge_index[0].reshape(NW, EPW)
    dst_r = edge_index[1].reshape(NW, NCHUNK, C)
    x_p = jnp.pad(x, ((0, NP - N), (0, 0)))
    ones_c = jnp.ones((RPS,), jnp.float32)
    zeros_c = jnp.zeros((C, D), jnp.float32)

    d0, d1 = _deg_kernel(dst_r, ones_c)
    g1, dinv = _tc1(d0.reshape(NP, 1), d1.reshape(NP, 1), x_p, W1)
    p0, p1 = _agg_kernel(g1, src_f, dst_r, zeros_c)
    g2 = _tc2(p0, p1, g1, dinv, b1.reshape(1, D), W2)
    q0, q1 = _agg_kernel(g2, src_f, dst_r, zeros_c)
    out = _tc3(q0, q1, g2, dinv, b2.reshape(1, D))
    return out[:N]
